# baseline (speedup 1.0000x reference)
import functools

import jax
import jax.numpy as jnp
from jax import lax
from jax.experimental import pallas as pl
from jax.experimental.pallas import tpu as pltpu
from jax.experimental.pallas import tpu_sc as plsc

N = 10000
E = 320000
D = 128
NC = 2
NS = 16
EPT = E // NS
K = 64
KC = 128
NB = 8
NCH = 320
NBLK = NCH // NB
NCHC = (NCH * K) // KC
NBLKC = NCHC // NB
EPAD = NCH * K
PF = 3
NRB = PF + 1
SRP = 640
NP = NS * SRP
CZ = 64

assert EPAD >= EPT and NCH % NB == 0 and SRP % CZ == 0

_mesh = plsc.VectorSubcoreMesh(
    core_axis_name="c", subcore_axis_name="s", num_cores=NC, num_subcores=NS
)


def _zero_stripe(zrow, buf, acc_sh, r0):
    cz = buf.shape[0]
    pltpu.sync_copy(zrow.at[pl.ds(0, cz)], buf)
    for z in range(SRP // cz):
        pltpu.sync_copy(buf, acc_sh.at[pl.ds(r0 + z * cz, cz)])


def _drain_stripe(acc_sh, buf, out, c, r0):
    cz = buf.shape[0]
    for z in range(SRP // cz):
        rz = r0 + z * cz
        pltpu.sync_copy(acc_sh.at[pl.ds(rz, cz)], buf)
        pltpu.sync_copy(buf, out.at[c, pl.ds(rz, cz)])


@functools.partial(
    pl.kernel, mesh=_mesh,
    out_type=jax.ShapeDtypeStruct((NC, NP, D), jnp.float32),
    scratch_types=[
        pltpu.VMEM((NB, K), jnp.int32),
        pltpu.VMEM((NB, K), jnp.int32),
        pltpu.VMEM((NB, K), jnp.int32),
        pltpu.VMEM((NB, K), jnp.int32),
        pltpu.VMEM((CZ, D), jnp.float32),
        [pltpu.VMEM((K, D), jnp.float32) for _ in range(NRB)],
        [pltpu.SemaphoreType.DMA for _ in range(NRB)],
        pltpu.VMEM_SHARED((NP, D), jnp.float32),
    ],
)
def _agg(table, dst5, src5, zrow, sums_out,
         dst_v0, src_v0, dst_v1, src_v1, bounce, rows, sems, acc_sh):
    c = lax.axis_index("c")
    s = lax.axis_index("s")
    r0 = s * SRP
    _zero_stripe(zrow, bounce, acc_sh, r0)
    plsc.subcore_barrier()

    tbl = table.at[c]
    idxs = ((dst_v0, src_v0), (dst_v1, src_v1))
    NH = NBLK // 2

    def fire(sv, b, t):
        pltpu.async_copy(tbl.at[sv.at[b]], rows[t % NRB], sems[t % NRB])

    pltpu.sync_copy(dst5.at[c, s * NBLK], dst_v0)
    pltpu.sync_copy(src5.at[c, s * NBLK], src_v0)
    for t in range(PF):
        fire(src_v0, t, t)

    def pair(j2, carry):
        w0 = s * NBLK + 2 * j2
        not_last = j2 < NH - 1
        pltpu.sync_copy(dst5.at[c, w0 + 1], dst_v1)
        pltpu.sync_copy(src5.at[c, w0 + 1], src_v1)
        for half, (dv, sv) in enumerate(idxs):
            if half == 1:
                @pl.when(not_last)
                def _():
                    pltpu.sync_copy(dst5.at[c, w0 + 2], dst_v0)
                    pltpu.sync_copy(src5.at[c, w0 + 2], src_v0)
            for b in range(NB):
                t = half * NB + b
                buf = rows[t % NRB]
                pltpu.make_async_copy(tbl.at[sv.at[b]], buf,
                                      sems[t % NRB]).wait()
                tgt = b + PF
                if tgt < NB:
                    fire(sv, tgt, t + PF)
                elif half == 0:
                    fire(src_v1, tgt - NB, t + PF)
                else:
                    @pl.when(not_last)
                    def _():
                        fire(src_v0, tgt - NB, t + PF)
                pltpu.sync_copy(buf, acc_sh.at[dv.at[b]], add=True)
        return carry

    lax.fori_loop(0, NH, pair, 0)
    plsc.subcore_barrier()
    _drain_stripe(acc_sh, bounce, sums_out, c, r0)


@functools.partial(
    pl.kernel, mesh=_mesh,
    out_type=jax.ShapeDtypeStruct((NC, NP, D), jnp.float32),
    scratch_types=[
        pltpu.VMEM((NB, KC), jnp.int32),
        pltpu.VMEM((KC, D), jnp.float32),
        pltpu.VMEM_SHARED((NP, D), jnp.float32),
    ],
)
def _cnt(dst5c, zrow, ones_h, cnts_out, dst_v, ones_v, acc_sh):
    c = lax.axis_index("c")
    s = lax.axis_index("s")
    r0 = s * SRP
    _zero_stripe(zrow, ones_v, acc_sh, r0)
    pltpu.sync_copy(ones_h, ones_v)
    plsc.subcore_barrier()

    def block(j, carry):
        w = s * NBLKC + j
        pltpu.sync_copy(dst5c.at[c, w], dst_v)
        for b in range(NB):
            pltpu.sync_copy(ones_v, acc_sh.at[dst_v.at[b]], add=True)
        return carry

    lax.fori_loop(0, NBLKC, block, 0)
    plsc.subcore_barrier()
    _drain_stripe(acc_sh, ones_v, cnts_out, c, r0)


def _layer(sums, cnts, Ws, bs, swap, concat):
    sel = (lambda j: (1 - j, 0, 0)) if swap else (lambda j: (j, 0, 0))
    if concat:
        out_shape = jax.ShapeDtypeStruct((2 * N, D), jnp.float32)
        out_spec = pl.BlockSpec((N, D), lambda j: (j, 0))
    else:
        out_shape = jax.ShapeDtypeStruct((2, N, D), jnp.float32)
        out_spec = pl.BlockSpec((1, N, D), lambda j: (j, 0, 0))

    def body(s_ref, c_ref, w_ref, b_ref, o_ref):
        cnt = jnp.maximum(c_ref[0][:N, 0:1], 1.0)
        mean = s_ref[0][:N] / cnt
        y = jnp.dot(mean, w_ref[0], preferred_element_type=jnp.float32)
        y = jnp.maximum(y + b_ref[0, 0], 0.0)
        if concat:
            o_ref[...] = y
        else:
            o_ref[0] = y

    return pl.pallas_call(
        body,
        grid=(2,),
        in_specs=[
            pl.BlockSpec((1, NP, D), sel),
            pl.BlockSpec((1, NP, D), sel),
            pl.BlockSpec((1, D, D), lambda j: (j, 0, 0)),
            pl.BlockSpec((1, 1, D), lambda j: (j, 0, 0)),
        ],
        out_specs=out_spec,
        out_shape=out_shape,
    )(sums, cnts, Ws, bs.reshape(2, 1, D))


def kernel(ft_a, ft_b, edge_ab, edge_ba, W0ab, b0ab, W0ba, b0ba,
           W1ab, b1ab, W1ba, b1ba):
    f32 = jnp.float32
    table1 = jnp.stack([ft_b, ft_a])
    dst = jnp.stack([edge_ab[0], edge_ba[0]]).astype(jnp.int32)
    src = jnp.stack([edge_ab[1], edge_ba[1]]).astype(jnp.int32)
    order = jnp.argsort(src, axis=1)
    src = jnp.take_along_axis(src, order, axis=1)
    dst = jnp.take_along_axis(dst, order, axis=1)
    pad = ((0, 0), (0, 0), (0, EPAD - EPT))
    dstp = jnp.pad(dst.reshape(NC, NS, EPT), pad, constant_values=N)
    dst5 = dstp.reshape(NC, NS * NBLK, NB, K)
    dst5c = dstp.reshape(NC, NS * NBLKC, NB, KC)
    src5 = jnp.pad(src.reshape(NC, NS, EPT), pad).reshape(NC, NS * NBLK, NB, K)
    zrow = jnp.zeros((KC, D), f32)
    ones_h = jnp.ones((KC, D), f32)

    cnts = _cnt(dst5c, zrow, ones_h)
    sums1 = _agg(table1, dst5, src5, zrow)
    W0s = jnp.stack([W0ba, W0ab])
    b0s = jnp.stack([b0ba, b0ab])
    embs1 = _layer(sums1, cnts, W0s, b0s, swap=True, concat=False)

    sums2 = _agg(embs1, dst5, src5, zrow)
    W1s = jnp.stack([W1ab, W1ba])
    b1s = jnp.stack([b1ab, b1ba])
    return _layer(sums2, cnts, W1s, b1s, swap=False, concat=True)


# speedup vs baseline: 3.4513x; 3.4513x over previous
import functools

import jax
import jax.numpy as jnp
from jax import lax
from jax.experimental import pallas as pl
from jax.experimental.pallas import tpu as pltpu
from jax.experimental.pallas import tpu_sc as plsc

N = 10000
E = 320000
D = 128
NC = 2
NS = 16
EPT = E // NS
K = 64
KC = 128
NB = 8
NCH = 320
NBLK = NCH // NB
NCHC = (NCH * K) // KC
NBLKC = NCHC // NB
EPAD = NCH * K
PF = 3
NRB = PF + 1
SRP = 640
NP = NS * SRP
CZ = 64

assert EPAD >= EPT and NCH % NB == 0 and SRP % CZ == 0

_mesh = plsc.VectorSubcoreMesh(
    core_axis_name="c", subcore_axis_name="s", num_cores=NC, num_subcores=NS
)


def _zero_stripe(zrow, buf, acc_sh, r0):
    cz = buf.shape[0]
    pltpu.sync_copy(zrow.at[pl.ds(0, cz)], buf)
    for z in range(SRP // cz):
        pltpu.sync_copy(buf, acc_sh.at[pl.ds(r0 + z * cz, cz)])


def _drain_stripe(acc_sh, buf, out, c, r0):
    cz = buf.shape[0]
    for z in range(SRP // cz):
        rz = r0 + z * cz
        pltpu.sync_copy(acc_sh.at[pl.ds(rz, cz)], buf)
        pltpu.sync_copy(buf, out.at[c, pl.ds(rz, cz)])


@functools.partial(
    pl.kernel, mesh=_mesh,
    out_type=jax.ShapeDtypeStruct((NC, NP, D), jnp.float32),
    scratch_types=[
        pltpu.VMEM((NB, K), jnp.int32),
        pltpu.VMEM((NB, K), jnp.int32),
        pltpu.VMEM((NB, K), jnp.int32),
        pltpu.VMEM((NB, K), jnp.int32),
        pltpu.VMEM((CZ, D), jnp.float32),
        [pltpu.VMEM((K, D), jnp.float32) for _ in range(NRB)],
        [pltpu.SemaphoreType.DMA for _ in range(NRB)],
        pltpu.VMEM_SHARED((NP, D), jnp.float32),
    ],
)
def _agg(table, dst5, src5, zrow, sums_out,
         dst_v0, src_v0, dst_v1, src_v1, bounce, rows, sems, acc_sh):
    c = lax.axis_index("c")
    s = lax.axis_index("s")
    r0 = s * SRP
    _zero_stripe(zrow, bounce, acc_sh, r0)
    plsc.subcore_barrier()

    tbl = table.at[c]
    idxs = ((dst_v0, src_v0), (dst_v1, src_v1))
    NH = NBLK // 2

    def fire(sv, b, t):
        pltpu.async_copy(tbl.at[sv.at[b]], rows[t % NRB], sems[t % NRB])

    pltpu.sync_copy(dst5.at[c, s * NBLK], dst_v0)
    pltpu.sync_copy(src5.at[c, s * NBLK], src_v0)
    for t in range(PF):
        fire(src_v0, t, t)

    def pair(j2, carry):
        w0 = s * NBLK + 2 * j2
        not_last = j2 < NH - 1
        pltpu.sync_copy(dst5.at[c, w0 + 1], dst_v1)
        pltpu.sync_copy(src5.at[c, w0 + 1], src_v1)
        for half, (dv, sv) in enumerate(idxs):
            if half == 1:
                @pl.when(not_last)
                def _():
                    pltpu.sync_copy(dst5.at[c, w0 + 2], dst_v0)
                    pltpu.sync_copy(src5.at[c, w0 + 2], src_v0)
            for b in range(NB):
                t = half * NB + b
                buf = rows[t % NRB]
                pltpu.make_async_copy(tbl.at[sv.at[b]], buf,
                                      sems[t % NRB]).wait()
                tgt = b + PF
                if tgt < NB:
                    fire(sv, tgt, t + PF)
                elif half == 0:
                    fire(src_v1, tgt - NB, t + PF)
                else:
                    @pl.when(not_last)
                    def _():
                        fire(src_v0, tgt - NB, t + PF)
                pltpu.sync_copy(buf, acc_sh.at[dv.at[b]], add=True)
        return carry

    lax.fori_loop(0, NH, pair, 0)
    plsc.subcore_barrier()
    _drain_stripe(acc_sh, bounce, sums_out, c, r0)


@functools.partial(
    pl.kernel, mesh=_mesh,
    out_type=jax.ShapeDtypeStruct((NC, NP, D), jnp.float32),
    scratch_types=[
        pltpu.VMEM((NB, KC), jnp.int32),
        pltpu.VMEM((KC, D), jnp.float32),
        pltpu.VMEM_SHARED((NP, D), jnp.float32),
        pltpu.SemaphoreType.DMA,
    ],
)
def _cnt(dst5c, zrow, ones_h, cnts_out, dst_v, ones_v, acc_sh, sem):
    c = lax.axis_index("c")
    s = lax.axis_index("s")
    r0 = s * SRP
    _zero_stripe(zrow, ones_v, acc_sh, r0)
    pltpu.sync_copy(ones_h, ones_v)
    plsc.subcore_barrier()

    def block(j, carry):
        w = s * NBLKC + j
        pltpu.sync_copy(dst5c.at[c, w], dst_v)
        for b in range(NB):
            pltpu.async_copy(ones_v, acc_sh.at[dst_v.at[b]], sem, add=True)
        for b in range(NB):
            pltpu.make_async_copy(ones_v, acc_sh.at[dst_v.at[b]], sem).wait()
        return carry

    lax.fori_loop(0, NBLKC, block, 0)
    plsc.subcore_barrier()
    _drain_stripe(acc_sh, ones_v, cnts_out, c, r0)


RB = 2000


def _layer(sums, cnts, Ws, bs, swap, concat):
    sel = ((lambda j, k: (1 - j, k, 0)) if swap
           else (lambda j, k: (j, k, 0)))
    if concat:
        out_shape = jax.ShapeDtypeStruct((2 * N, D), jnp.float32)
        out_spec = pl.BlockSpec((RB, D), lambda j, k: (j * (N // RB) + k, 0))
    else:
        out_shape = jax.ShapeDtypeStruct((2, N, D), jnp.float32)
        out_spec = pl.BlockSpec((1, RB, D), lambda j, k: (j, k, 0))

    def body(s_ref, c_ref, w_ref, b_ref, o_ref):
        cnt = jnp.maximum(c_ref[0][:, 0:1], 1.0)
        mean = s_ref[0] / cnt
        y = jnp.dot(mean, w_ref[0], preferred_element_type=jnp.float32)
        y = jnp.maximum(y + b_ref[0, 0], 0.0)
        if concat:
            o_ref[...] = y
        else:
            o_ref[0] = y

    return pl.pallas_call(
        body,
        grid=(2, N // RB),
        in_specs=[
            pl.BlockSpec((1, RB, D), sel),
            pl.BlockSpec((1, RB, D), sel),
            pl.BlockSpec((1, D, D), lambda j, k: (j, 0, 0)),
            pl.BlockSpec((1, 1, D), lambda j, k: (j, 0, 0)),
        ],
        out_specs=out_spec,
        out_shape=out_shape,
    )(sums, cnts, Ws, bs.reshape(2, 1, D))


def kernel(ft_a, ft_b, edge_ab, edge_ba, W0ab, b0ab, W0ba, b0ba,
           W1ab, b1ab, W1ba, b1ba):
    f32 = jnp.float32
    table1 = jnp.stack([ft_b, ft_a])
    dst = jnp.stack([edge_ab[0], edge_ba[0]]).astype(jnp.int32)
    src = jnp.stack([edge_ab[1], edge_ba[1]]).astype(jnp.int32)
    pad = ((0, 0), (0, 0), (0, EPAD - EPT))
    dstp = jnp.pad(dst.reshape(NC, NS, EPT), pad, constant_values=N)
    dst5 = dstp.reshape(NC, NS * NBLK, NB, K)
    dst5c = dstp.reshape(NC, NS * NBLKC, NB, KC)
    src5 = jnp.pad(src.reshape(NC, NS, EPT), pad).reshape(NC, NS * NBLK, NB, K)
    zrow = jnp.zeros((KC, D), f32)
    ones_h = jnp.ones((KC, D), f32)

    cnts = _cnt(dst5c, zrow, ones_h)
    sums1 = _agg(table1, dst5, src5, zrow)
    W0s = jnp.stack([W0ba, W0ab])
    b0s = jnp.stack([b0ba, b0ab])
    embs1 = _layer(sums1, cnts, W0s, b0s, swap=True, concat=False)

    sums2 = _agg(embs1, dst5, src5, zrow)
    W1s = jnp.stack([W1ab, W1ba])
    b1s = jnp.stack([b1ab, b1ba])
    return _layer(sums2, cnts, W1s, b1s, swap=False, concat=True)


# speedup vs baseline: 3.4958x; 1.0129x over previous
import functools

import jax
import jax.numpy as jnp
from jax import lax
from jax.experimental import pallas as pl
from jax.experimental.pallas import tpu as pltpu
from jax.experimental.pallas import tpu_sc as plsc

N = 10000
E = 320000
D = 128
NC = 2
NS = 16
EPT = E // NS
K = 64
KC = 128
NB = 8
NCH = 320
NBLK = NCH // NB
NCHC = (NCH * K) // KC
NBLKC = NCHC // NB
EPAD = NCH * K
PF = 3
NRB = PF + 1
SRP = 640
NP = NS * SRP
CZ = 64

assert EPAD >= EPT and NCH % NB == 0 and SRP % CZ == 0

_mesh = plsc.VectorSubcoreMesh(
    core_axis_name="c", subcore_axis_name="s", num_cores=NC, num_subcores=NS
)


def _zero_stripe(zrow, buf, acc_sh, r0):
    cz = buf.shape[0]
    pltpu.sync_copy(zrow.at[pl.ds(0, cz)], buf)
    for z in range(SRP // cz):
        pltpu.sync_copy(buf, acc_sh.at[pl.ds(r0 + z * cz, cz)])


def _drain_stripe(acc_sh, buf, out, c, r0):
    cz = buf.shape[0]
    for z in range(SRP // cz):
        rz = r0 + z * cz
        pltpu.sync_copy(acc_sh.at[pl.ds(rz, cz)], buf)
        pltpu.sync_copy(buf, out.at[c, pl.ds(rz, cz)])


@functools.partial(
    pl.kernel, mesh=_mesh,
    out_type=jax.ShapeDtypeStruct((NC, NP, D), jnp.float32),
    scratch_types=[
        pltpu.VMEM((NB, K), jnp.int32),
        pltpu.VMEM((NB, K), jnp.int32),
        pltpu.VMEM((NB, K), jnp.int32),
        pltpu.VMEM((NB, K), jnp.int32),
        pltpu.VMEM((CZ, D), jnp.float32),
        [pltpu.VMEM((K, D), jnp.float32) for _ in range(NRB)],
        [pltpu.SemaphoreType.DMA for _ in range(NRB)],
        [pltpu.SemaphoreType.DMA for _ in range(NRB)],
        pltpu.VMEM_SHARED((NP, D), jnp.float32),
    ],
)
def _agg(table, dst5, src5, zrow, sums_out,
         dst_v0, src_v0, dst_v1, src_v1, bounce, rows, sems, ssems, acc_sh):
    c = lax.axis_index("c")
    s = lax.axis_index("s")
    r0 = s * SRP
    _zero_stripe(zrow, bounce, acc_sh, r0)
    plsc.subcore_barrier()

    tbl = table.at[c]
    idxs = ((dst_v0, src_v0), (dst_v1, src_v1))
    NH = NBLK // 2

    def fire(sv, b, t):
        pltpu.async_copy(tbl.at[sv.at[b]], rows[t % NRB], sems[t % NRB])

    pltpu.sync_copy(dst5.at[c, s * NBLK], dst_v0)
    pltpu.sync_copy(src5.at[c, s * NBLK], src_v0)
    for t in range(PF):
        fire(src_v0, t, t)

    def pair(j2, carry):
        w0 = s * NBLK + 2 * j2
        not_last = j2 < NH - 1
        pltpu.sync_copy(dst5.at[c, w0 + 1], dst_v1)
        pltpu.sync_copy(src5.at[c, w0 + 1], src_v1)
        for half, (dv, sv) in enumerate(idxs):
            if half == 1:
                @pl.when(not_last)
                def _():
                    pltpu.sync_copy(dst5.at[c, w0 + 2], dst_v0)
                    pltpu.sync_copy(src5.at[c, w0 + 2], src_v0)
            for b in range(NB):
                t = half * NB + b
                i = t % NRB
                ip = (t + PF) % NRB
                buf = rows[i]
                pltpu.make_async_copy(tbl.at[sv.at[b]], buf, sems[i]).wait()

                def _refire():
                    pltpu.make_async_copy(rows[ip], acc_sh.at[dv.at[b]],
                                          ssems[ip]).wait()
                    tgt = b + PF
                    if tgt < NB:
                        fire(sv, tgt, t + PF)
                    elif half == 0:
                        fire(src_v1, tgt - NB, t + PF)
                    else:
                        @pl.when(not_last)
                        def _():
                            fire(src_v0, tgt - NB, t + PF)

                if t == 0:
                    @pl.when(j2 > 0)
                    def _():
                        _refire()

                    @pl.when(j2 == 0)
                    def _():
                        tgt = b + PF
                        fire(sv, tgt, t + PF)
                else:
                    _refire()
                pltpu.async_copy(buf, acc_sh.at[dv.at[b]], ssems[i], add=True)
        return carry

    lax.fori_loop(0, NH, pair, 0)
    ilast = (2 * NB - 1) % NRB
    pltpu.make_async_copy(rows[ilast], acc_sh.at[dst_v1.at[0]],
                          ssems[ilast]).wait()
    plsc.subcore_barrier()
    _drain_stripe(acc_sh, bounce, sums_out, c, r0)


@functools.partial(
    pl.kernel, mesh=_mesh,
    out_type=jax.ShapeDtypeStruct((NC, NP, D), jnp.float32),
    scratch_types=[
        pltpu.VMEM((NB, KC), jnp.int32),
        pltpu.VMEM((KC, D), jnp.float32),
        pltpu.VMEM_SHARED((NP, D), jnp.float32),
        pltpu.SemaphoreType.DMA,
    ],
)
def _cnt(dst5c, zrow, ones_h, cnts_out, dst_v, ones_v, acc_sh, sem):
    c = lax.axis_index("c")
    s = lax.axis_index("s")
    r0 = s * SRP
    _zero_stripe(zrow, ones_v, acc_sh, r0)
    pltpu.sync_copy(ones_h, ones_v)
    plsc.subcore_barrier()

    def block(j, carry):
        w = s * NBLKC + j
        pltpu.sync_copy(dst5c.at[c, w], dst_v)
        for b in range(NB):
            pltpu.async_copy(ones_v, acc_sh.at[dst_v.at[b]], sem, add=True)
        for b in range(NB):
            pltpu.make_async_copy(ones_v, acc_sh.at[dst_v.at[b]], sem).wait()
        return carry

    lax.fori_loop(0, NBLKC, block, 0)
    plsc.subcore_barrier()
    _drain_stripe(acc_sh, ones_v, cnts_out, c, r0)


RB = 2000


def _layer(sums, cnts, Ws, bs, swap, concat):
    sel = ((lambda j, k: (1 - j, k, 0)) if swap
           else (lambda j, k: (j, k, 0)))
    if concat:
        out_shape = jax.ShapeDtypeStruct((2 * N, D), jnp.float32)
        out_spec = pl.BlockSpec((RB, D), lambda j, k: (j * (N // RB) + k, 0))
    else:
        out_shape = jax.ShapeDtypeStruct((2, N, D), jnp.float32)
        out_spec = pl.BlockSpec((1, RB, D), lambda j, k: (j, k, 0))

    def body(s_ref, c_ref, w_ref, b_ref, o_ref):
        cnt = jnp.maximum(c_ref[0][:, 0:1], 1.0)
        mean = s_ref[0] / cnt
        y = jnp.dot(mean, w_ref[0], preferred_element_type=jnp.float32)
        y = jnp.maximum(y + b_ref[0, 0], 0.0)
        if concat:
            o_ref[...] = y
        else:
            o_ref[0] = y

    return pl.pallas_call(
        body,
        grid=(2, N // RB),
        in_specs=[
            pl.BlockSpec((1, RB, D), sel),
            pl.BlockSpec((1, RB, D), sel),
            pl.BlockSpec((1, D, D), lambda j, k: (j, 0, 0)),
            pl.BlockSpec((1, 1, D), lambda j, k: (j, 0, 0)),
        ],
        out_specs=out_spec,
        out_shape=out_shape,
    )(sums, cnts, Ws, bs.reshape(2, 1, D))


def kernel(ft_a, ft_b, edge_ab, edge_ba, W0ab, b0ab, W0ba, b0ba,
           W1ab, b1ab, W1ba, b1ba):
    f32 = jnp.float32
    table1 = jnp.stack([ft_b, ft_a])
    dst = jnp.stack([edge_ab[0], edge_ba[0]]).astype(jnp.int32)
    src = jnp.stack([edge_ab[1], edge_ba[1]]).astype(jnp.int32)
    pad = ((0, 0), (0, 0), (0, EPAD - EPT))
    dstp = jnp.pad(dst.reshape(NC, NS, EPT), pad, constant_values=N)
    dst5 = dstp.reshape(NC, NS * NBLK, NB, K)
    dst5c = dstp.reshape(NC, NS * NBLKC, NB, KC)
    src5 = jnp.pad(src.reshape(NC, NS, EPT), pad).reshape(NC, NS * NBLK, NB, K)
    zrow = jnp.zeros((KC, D), f32)
    ones_h = jnp.ones((KC, D), f32)

    cnts = _cnt(dst5c, zrow, ones_h)
    sums1 = _agg(table1, dst5, src5, zrow)
    W0s = jnp.stack([W0ba, W0ab])
    b0s = jnp.stack([b0ba, b0ab])
    embs1 = _layer(sums1, cnts, W0s, b0s, swap=True, concat=False)

    sums2 = _agg(embs1, dst5, src5, zrow)
    W1s = jnp.stack([W1ab, W1ba])
    b1s = jnp.stack([b1ab, b1ba])
    return _layer(sums2, cnts, W1s, b1s, swap=False, concat=True)
